# baseline (speedup 1.0000x reference)
import functools

import jax
import jax.numpy as jnp
from jax import lax
from jax.experimental import pallas as pl
from jax.experimental.pallas import tpu as pltpu
from jax.experimental.pallas import tpu_sc as plsc

NUM_TOKENS = 16384
EMBED_DIM = 32
NUM_TAGS = 32

_info = plsc.get_sparse_core_info()
_NC, _NS = _info.num_cores, _info.num_subcores
_NW = _NC * _NS
_BPW = NUM_TOKENS // _NW
_ICH = 128
_NCH = _BPW // _ICH


def _make_gather():
  mesh = plsc.VectorSubcoreMesh(core_axis_name="c", subcore_axis_name="s")

  @functools.partial(
      pl.kernel,
      mesh=mesh,
      compiler_params=pltpu.CompilerParams(use_tc_tiling_on_sc=False),
      out_type=jax.ShapeDtypeStruct((NUM_TOKENS, EMBED_DIM), jnp.float32),
      scratch_types=[
          pltpu.VMEM((_NCH, _ICH), jnp.int32),
          pltpu.VMEM((_BPW, EMBED_DIM), jnp.float32),
          pltpu.SemaphoreType.DMA,
      ],
  )
  def gather_k(idx_hbm, table_hbm, out_hbm, idx_v, rows_v, sem):
    wid = lax.axis_index("s") * _NC + lax.axis_index("c")
    base = wid * _BPW
    pltpu.sync_copy(idx_hbm.at[wid], idx_v)
    copies = []
    for j in range(_NCH):
      copies.append(
          pltpu.async_copy(
              table_hbm.at[idx_v.at[j]],
              rows_v.at[pl.ds(j * _ICH, _ICH)],
              sem,
          ))
    for c in copies:
      c.wait()
    pltpu.sync_copy(rows_v, out_hbm.at[pl.ds(base, _BPW)])

  return gather_k


_gather = _make_gather()


def _mm_body(e_ref, w_ref, b_ref, o_ref):
  o_ref[...] = (
      lax.dot_general(e_ref[...], w_ref[...], (((1,), (1,)), ((), ())),
                      preferred_element_type=jnp.float32)
      + b_ref[...])


_BM = 2048


def _matmul(embs, W, b2d):
  return pl.pallas_call(
      _mm_body,
      grid=(NUM_TOKENS // _BM,),
      in_specs=[
          pl.BlockSpec((_BM, EMBED_DIM), lambda i: (i, 0)),
          pl.BlockSpec((NUM_TAGS, EMBED_DIM), lambda i: (0, 0)),
          pl.BlockSpec((1, NUM_TAGS), lambda i: (0, 0)),
      ],
      out_specs=pl.BlockSpec((_BM, NUM_TAGS), lambda i: (i, 0)),
      out_shape=jax.ShapeDtypeStruct((NUM_TOKENS, NUM_TAGS), jnp.float32),
  )(embs, W, b2d)


def kernel(sent, emb_table, W, b):
  idx = sent.reshape(_NW, _NCH, _ICH)
  embs = _gather(idx, emb_table)
  return _matmul(embs, W, b.reshape(1, NUM_TAGS))


# speedup vs baseline: 1.6404x; 1.6404x over previous
import functools

import jax
import jax.numpy as jnp
from jax import lax
from jax.experimental import pallas as pl
from jax.experimental.pallas import tpu as pltpu
from jax.experimental.pallas import tpu_sc as plsc

NUM_TOKENS = 16384
EMBED_DIM = 32
NUM_TAGS = 32

_info = plsc.get_sparse_core_info()
_NC, _NS = _info.num_cores, _info.num_subcores
_NW = _NC * _NS
_BPW = NUM_TOKENS // _NW
_ICH = 128
_NCH = _BPW // _ICH


def _make_gather():
  mesh = plsc.VectorSubcoreMesh(core_axis_name="c", subcore_axis_name="s")

  @functools.partial(
      pl.kernel,
      mesh=mesh,
      out_type=jax.ShapeDtypeStruct((NUM_TOKENS, EMBED_DIM), jnp.float32),
      scratch_types=[
          pltpu.VMEM((_BPW,), jnp.int32),
          pltpu.VMEM((_BPW, EMBED_DIM), jnp.float32),
          pltpu.SemaphoreType.DMA,
      ],
  )
  def gather_k(idx_hbm, table_hbm, out_hbm, idx_v, rows_v, sem):
    wid = lax.axis_index("s") * _NC + lax.axis_index("c")
    base = wid * _BPW
    pltpu.sync_copy(idx_hbm.at[pl.ds(base, _BPW)], idx_v)

    def body(g, carry):
      vec = idx_v[pl.ds(g * 16, 16)]
      for j in range(16):
        r = lax.squeeze(lax.slice(vec, (j,), (j + 1,)), (0,))
        pltpu.async_copy(
            table_hbm.at[pl.ds(r, 1)], rows_v.at[pl.ds(g * 16 + j, 1)], sem)
      return carry

    lax.fori_loop(0, _BPW // 16, body, 0)
    pltpu.make_async_copy(table_hbm.at[pl.ds(0, _BPW)], rows_v, sem).wait()
    pltpu.sync_copy(rows_v, out_hbm.at[pl.ds(base, _BPW)])

  return gather_k


_gather = _make_gather()


def _mm_body(e_ref, w_ref, b_ref, o_ref):
  o_ref[...] = (
      lax.dot_general(e_ref[...], w_ref[...], (((1,), (1,)), ((), ())),
                      preferred_element_type=jnp.float32)
      + b_ref[...])


_BM = 2048


def _matmul(embs, W, b2d):
  return pl.pallas_call(
      _mm_body,
      grid=(NUM_TOKENS // _BM,),
      in_specs=[
          pl.BlockSpec((_BM, EMBED_DIM), lambda i: (i, 0)),
          pl.BlockSpec((NUM_TAGS, EMBED_DIM), lambda i: (0, 0)),
          pl.BlockSpec((1, NUM_TAGS), lambda i: (0, 0)),
      ],
      out_specs=pl.BlockSpec((_BM, NUM_TAGS), lambda i: (i, 0)),
      out_shape=jax.ShapeDtypeStruct((NUM_TOKENS, NUM_TAGS), jnp.float32),
  )(embs, W, b2d)


def kernel(sent, emb_table, W, b):
  embs = _gather(sent, emb_table)
  return _matmul(embs, W, b.reshape(1, NUM_TAGS))


# speedup vs baseline: 1.9864x; 1.2109x over previous
import functools

import jax
import jax.numpy as jnp
from jax import lax
from jax.experimental import pallas as pl
from jax.experimental.pallas import tpu as pltpu
from jax.experimental.pallas import tpu_sc as plsc

NUM_EMB = 1000000
NUM_TOKENS = 16384
EMBED_DIM = 32
NUM_TAGS = 32

_info = plsc.get_sparse_core_info()
_NC, _NS = _info.num_cores, _info.num_subcores
_NW = _NC * _NS
_BPW = NUM_TOKENS // _NW

_TN = 8192


def _tr_body(tT_ref, o_ref):
  o_ref[...] = jnp.swapaxes(tT_ref[...], 0, 1)


def _transpose_table(tableT):
  return pl.pallas_call(
      _tr_body,
      grid=(pl.cdiv(NUM_EMB, _TN),),
      in_specs=[pl.BlockSpec((EMBED_DIM, _TN), lambda i: (0, i))],
      out_specs=pl.BlockSpec((_TN, EMBED_DIM), lambda i: (i, 0)),
      out_shape=jax.ShapeDtypeStruct((NUM_EMB, EMBED_DIM), jnp.float32),
  )(tableT)


def _make_gather():
  mesh = plsc.VectorSubcoreMesh(core_axis_name="c", subcore_axis_name="s")

  @functools.partial(
      pl.kernel,
      mesh=mesh,
      out_type=jax.ShapeDtypeStruct((NUM_TOKENS, EMBED_DIM), jnp.float32),
      scratch_types=[
          pltpu.VMEM((_BPW,), jnp.int32),
          pltpu.VMEM((_BPW, EMBED_DIM), jnp.float32),
          pltpu.SemaphoreType.DMA,
      ],
  )
  def gather_k(idx_hbm, table_hbm, out_hbm, idx_v, rows_v, sem):
    wid = lax.axis_index("s") * _NC + lax.axis_index("c")
    base = wid * _BPW
    pltpu.sync_copy(idx_hbm.at[pl.ds(base, _BPW)], idx_v)

    def body(g, carry):
      vec = idx_v[pl.ds(g * 16, 16)]
      for j in range(16):
        r = lax.squeeze(lax.slice(vec, (j,), (j + 1,)), (0,))
        pltpu.async_copy(
            table_hbm.at[pl.ds(r, 1)], rows_v.at[pl.ds(g * 16 + j, 1)], sem)
      return carry

    lax.fori_loop(0, _BPW // 16, body, 0)
    pltpu.make_async_copy(table_hbm.at[pl.ds(0, _BPW)], rows_v, sem).wait()
    pltpu.sync_copy(rows_v, out_hbm.at[pl.ds(base, _BPW)])

  return gather_k


_gather = _make_gather()


_BM = 2048


def _mm_body(w_ref, b_ref, e_ref, o_ref):
  o_ref[...] = (
      lax.dot_general(w_ref[...], e_ref[...], (((1,), (1,)), ((), ())),
                      preferred_element_type=jnp.float32)
      + b_ref[...])


def _matmul(W, b2d, embs):
  return pl.pallas_call(
      _mm_body,
      grid=(NUM_TOKENS // _BM,),
      in_specs=[
          pl.BlockSpec((NUM_TAGS, EMBED_DIM), lambda i: (0, 0)),
          pl.BlockSpec((NUM_TAGS, 1), lambda i: (0, 0)),
          pl.BlockSpec((_BM, EMBED_DIM), lambda i: (i, 0)),
      ],
      out_specs=pl.BlockSpec((NUM_TAGS, _BM), lambda i: (0, i)),
      out_shape=jax.ShapeDtypeStruct((NUM_TAGS, NUM_TOKENS), jnp.float32),
  )(W, b2d, embs)


def kernel(sent, emb_table, W, b):
  tableT = jnp.swapaxes(emb_table, 0, 1)
  table_rm = _transpose_table(tableT)
  embs = _gather(sent, table_rm)
  scoresT = _matmul(W, b.reshape(NUM_TAGS, 1), embs)
  return jnp.swapaxes(scoresT, 0, 1)
